# baseline (speedup 1.0000x reference)
import jax
import jax.numpy as jnp
from jax.experimental import pallas as pl
from jax.experimental.pallas import tpu as pltpu

B, S, D, K = 4, 4096, 4096, 8
BS = 512
NS = S // BS
NEG_INF = float("-inf")


def _scores_topk_kernel(x_ref, w_ref, b_ref, idx_ref, s_scratch):
    j = pl.program_id(1)
    xb = x_ref[0].astype(jnp.bfloat16)
    wb = w_ref[...].astype(jnp.bfloat16)
    part = jax.lax.dot_general(
        xb, wb,
        dimension_numbers=(((1,), (0,)), ((), ())),
        preferred_element_type=jnp.float32)
    s_scratch[0, pl.ds(j * BS, BS)] = part[:, 0] + b_ref[0, 0]

    @pl.when(j == NS - 1)
    def _():
        iota = jax.lax.broadcasted_iota(jnp.int32, (1, S), 1)
        kio = jax.lax.broadcasted_iota(jnp.int32, (1, K), 1)
        sv = s_scratch[0, :][None, :]
        idxs = jnp.zeros((1, K), jnp.int32)
        for k in range(K):
            m = jnp.max(sv)
            idx = jnp.min(jnp.where(sv == m, iota, S))
            idxs = jnp.where(kio == k, idx, idxs)
            sv = jnp.where(iota == idx, NEG_INF, sv)
        idx_ref[...] = idxs.reshape(1, 1, K)


def _gather_kernel(idx_flat_ref, x_ref, out_ref):
    del idx_flat_ref
    out_ref[...] = x_ref[...]


def kernel(x, W, b):
    idx3 = pl.pallas_call(
        _scores_topk_kernel,
        grid=(B, NS),
        in_specs=[
            pl.BlockSpec((1, BS, D), lambda bi, j: (bi, j, 0)),
            pl.BlockSpec((D, 1), lambda bi, j: (0, 0)),
            pl.BlockSpec((1, 1), lambda bi, j: (0, 0)),
        ],
        out_specs=pl.BlockSpec((1, 1, K), lambda bi, j: (bi, 0, 0)),
        out_shape=jax.ShapeDtypeStruct((B, 1, K), jnp.int32),
        scratch_shapes=[pltpu.VMEM((1, S), jnp.float32)],
        compiler_params=pltpu.CompilerParams(
            dimension_semantics=("parallel", "arbitrary"),
        ),
    )(x, W.T, b.reshape(1, 1))

    top_k_indices = idx3.reshape(B, K)
    idx_flat = (top_k_indices + jnp.arange(B, dtype=jnp.int32)[:, None] * S
                ).reshape(B * K)

    x3d = x.reshape(B * S, 1, D)
    gathered = pl.pallas_call(
        _gather_kernel,
        grid_spec=pltpu.PrefetchScalarGridSpec(
            num_scalar_prefetch=1,
            grid=(B * K,),
            in_specs=[pl.BlockSpec((1, 1, D), lambda i, idx: (idx[i], 0, 0))],
            out_specs=pl.BlockSpec((1, 1, D), lambda i, idx: (i, 0, 0)),
        ),
        out_shape=jax.ShapeDtypeStruct((B * K, 1, D), jnp.float32),
    )(idx_flat, x3d)

    x_top_k = gathered.reshape(B, K, D)
    return (x_top_k, top_k_indices[:, :, None])


# speedup vs baseline: 2.4054x; 2.4054x over previous
import jax
import jax.numpy as jnp
from jax.experimental import pallas as pl
from jax.experimental.pallas import tpu as pltpu

B, S, D, K = 4, 4096, 4096, 8
BS = 512
NS = S // BS
NEG_INF = float("-inf")


def _scores_topk_kernel(x_ref, w_ref, b_ref, idx_ref, s_scratch):
    j = pl.program_id(1)
    xb = x_ref[0].astype(jnp.bfloat16).astype(jnp.float32)
    wb = w_ref[...].astype(jnp.bfloat16).astype(jnp.float32)
    v = xb * wb
    w_ = D
    while w_ > 128:
        h = w_ // 2
        v = v[:, :h] + v[:, h:w_]
        w_ = h
    part = jnp.sum(v, axis=1)
    s_scratch[0, pl.ds(j * BS, BS)] = part + b_ref[0, 0]

    @pl.when(j == NS - 1)
    def _():
        iota = jax.lax.broadcasted_iota(jnp.int32, (1, S), 1)
        kio = jax.lax.broadcasted_iota(jnp.int32, (1, K), 1)
        sv = s_scratch[0, :][None, :]
        idxs = jnp.zeros((1, K), jnp.int32)
        for k in range(K):
            m = jnp.max(sv)
            idx = jnp.min(jnp.where(sv == m, iota, S))
            idxs = jnp.where(kio == k, idx, idxs)
            sv = jnp.where(iota == idx, NEG_INF, sv)
        idx_ref[...] = idxs.reshape(1, 1, K)


def _gather_kernel(idx_flat_ref, x_ref, out_ref):
    del idx_flat_ref
    out_ref[...] = x_ref[0]


def kernel(x, W, b):
    idx3 = pl.pallas_call(
        _scores_topk_kernel,
        grid=(B, NS),
        in_specs=[
            pl.BlockSpec((1, BS, D), lambda bi, j: (bi, j, 0)),
            pl.BlockSpec((1, D), lambda bi, j: (0, 0)),
            pl.BlockSpec((1, 1), lambda bi, j: (0, 0)),
        ],
        out_specs=pl.BlockSpec((1, 1, K), lambda bi, j: (bi, 0, 0)),
        out_shape=jax.ShapeDtypeStruct((B, 1, K), jnp.int32),
        scratch_shapes=[pltpu.VMEM((1, S), jnp.float32)],
        compiler_params=pltpu.CompilerParams(
            dimension_semantics=("parallel", "arbitrary"),
        ),
    )(x, W, b.reshape(1, 1))

    top_k_indices = idx3.reshape(B, K)
    idx_rows = top_k_indices.reshape(B * K)

    x4d = x.reshape(B, S, 1, D)
    gathered = pl.pallas_call(
        _gather_kernel,
        grid_spec=pltpu.PrefetchScalarGridSpec(
            num_scalar_prefetch=1,
            grid=(B * K,),
            in_specs=[pl.BlockSpec((1, 1, 1, D),
                                   lambda i, idx: (i // K, idx[i], 0, 0))],
            out_specs=pl.BlockSpec((1, 1, D), lambda i, idx: (i, 0, 0)),
        ),
        out_shape=jax.ShapeDtypeStruct((B * K, 1, D), jnp.float32),
    )(idx_rows, x4d)

    x_top_k = gathered.reshape(B, K, D)
    return (x_top_k, top_k_indices[:, :, None])


# speedup vs baseline: 8.8617x; 3.6841x over previous
import jax
import jax.numpy as jnp
from jax.experimental import pallas as pl
from jax.experimental.pallas import tpu as pltpu

B, S, D, K = 4, 4096, 4096, 8
BS = 512
NS = S // BS
NEG_INF = float("-inf")


def _scores_topk_kernel(x_ref, w_ref, b_ref, idx_ref, s_scratch):
    j = pl.program_id(1)
    xb = x_ref[0].astype(jnp.bfloat16).astype(jnp.float32)
    wb = w_ref[...].astype(jnp.bfloat16).astype(jnp.float32)
    v = xb * wb
    w_ = D
    while w_ > 128:
        h = w_ // 2
        v = v[:, :h] + v[:, h:w_]
        w_ = h
    part = jnp.sum(v, axis=1)
    s_scratch[0, pl.ds(j * BS, BS)] = part + b_ref[0, 0]

    @pl.when(j == NS - 1)
    def _():
        iota = jax.lax.broadcasted_iota(jnp.int32, (1, S), 1)
        kio = jax.lax.broadcasted_iota(jnp.int32, (1, K), 1)
        sv = s_scratch[0, :][None, :]
        idxs = jnp.zeros((1, K), jnp.int32)
        for k in range(K):
            m = jnp.max(sv)
            idx = jnp.min(jnp.where(sv == m, iota, S))
            idxs = jnp.where(kio == k, idx, idxs)
            sv = jnp.where(iota == idx, NEG_INF, sv)
        idx_ref[...] = idxs.reshape(1, 1, K)


def _gather_kernel(idx_ref, x_ref, out_ref, sems):
    copies = []
    for i in range(B * K):
        bi, k = i // K, i % K
        row = idx_ref[i]
        copies.append(pltpu.make_async_copy(
            x_ref.at[bi, pl.ds(row, 1), :],
            out_ref.at[bi, pl.ds(k, 1), :],
            sems.at[i]))
    for c in copies:
        c.start()
    for c in copies:
        c.wait()


def kernel(x, W, b):
    idx3 = pl.pallas_call(
        _scores_topk_kernel,
        grid=(B, NS),
        in_specs=[
            pl.BlockSpec((1, BS, D), lambda bi, j: (bi, j, 0)),
            pl.BlockSpec((1, D), lambda bi, j: (0, 0)),
            pl.BlockSpec((1, 1), lambda bi, j: (0, 0)),
        ],
        out_specs=pl.BlockSpec((1, 1, K), lambda bi, j: (bi, 0, 0)),
        out_shape=jax.ShapeDtypeStruct((B, 1, K), jnp.int32),
        scratch_shapes=[pltpu.VMEM((1, S), jnp.float32)],
        compiler_params=pltpu.CompilerParams(
            dimension_semantics=("parallel", "arbitrary"),
        ),
    )(x, W, b.reshape(1, 1))

    top_k_indices = idx3.reshape(B, K)
    idx_rows = top_k_indices.reshape(B * K)

    x_top_k = pl.pallas_call(
        _gather_kernel,
        grid_spec=pltpu.PrefetchScalarGridSpec(
            num_scalar_prefetch=1,
            grid=(1,),
            in_specs=[pl.BlockSpec(memory_space=pl.ANY)],
            out_specs=pl.BlockSpec(memory_space=pl.ANY),
            scratch_shapes=[pltpu.SemaphoreType.DMA((B * K,))],
        ),
        out_shape=jax.ShapeDtypeStruct((B, K, D), jnp.float32),
    )(idx_rows, x)

    return (x_top_k, top_k_indices[:, :, None])
